# baseline (speedup 1.0000x reference)
import jax
import jax.numpy as jnp
from jax import lax
from jax.experimental import pallas as pl
from jax.experimental.pallas import tpu as pltpu

N_ROIS = 20000
LOSS_DIM = 84
KEEP = 512
VR = 13125
VC = 128
PH = 21
GR = VR // PH
SEGS = 2688 // LOSS_DIM
F32_INF_BITS = 0x7F800000


def _ohem_body(t_ref, p_ref, out_ref):
    d = jnp.abs(t_ref[...] - p_ref[...])
    l = jnp.where(d < 1.0, 0.5 * d * d, d - 0.5)
    l3 = jnp.reshape(l, (GR, PH, VC))

    cc = lax.broadcasted_iota(jnp.int32, (VC, SEGS), 0)
    ss = lax.broadcasted_iota(jnp.int32, (VC, SEGS), 1)
    vals = jnp.zeros((GR, SEGS), jnp.float32)
    for p in range(PH):
        bp = jnp.where((p * VC + cc) // LOSS_DIM == ss, 1.0, 0.0)
        vals = vals + lax.dot_general(
            l3[:, p, :], bp.astype(jnp.float32),
            dimension_numbers=(((1,), (0,)), ((), ())),
            precision=lax.Precision.DEFAULT,
            preferred_element_type=jnp.float32,
        )
    bits = lax.bitcast_convert_type(vals, jnp.int32)

    def count_ge(m):
        return jnp.sum(jnp.where(bits >= m, 1, 0))

    def body(_, carry):
        lo, hi = carry
        q = jnp.maximum((hi - lo) // 4, 1)
        m1 = lo + q
        m2 = lo + 2 * q
        m3 = lo + 3 * q
        c1 = count_ge(m1) >= KEEP
        c2 = count_ge(m2) >= KEEP
        c3 = count_ge(m3) >= KEEP
        lo2 = jnp.where(c3, m3, jnp.where(c2, m2, jnp.where(c1, m1, lo)))
        hi2 = jnp.where(c1, jnp.where(c2, jnp.where(c3, hi, m3), m2), m1)
        return lo2, hi2

    lo, hi = lax.fori_loop(
        0, 16, body, (jnp.int32(0), jnp.int32(F32_INF_BITS)))
    t_val = lax.bitcast_convert_type(lo, jnp.float32)
    gt = bits > lo
    cnt_gt = jnp.sum(jnp.where(gt, 1, 0))
    sum_gt = jnp.sum(jnp.where(gt, vals, 0.0))
    res = (sum_gt + (KEEP - cnt_gt).astype(jnp.float32) * t_val) / KEEP
    out_ref[0, 0] = res


@jax.jit
def _ohem(target, predict):
    tv = target.reshape(VR, VC)
    pv = predict.reshape(VR, VC)
    out = pl.pallas_call(
        _ohem_body,
        out_specs=pl.BlockSpec(memory_space=pltpu.SMEM),
        out_shape=jax.ShapeDtypeStruct((1, 1), jnp.float32),
    )(tv, pv)
    return out[0, 0]


def kernel(target, predict):
    return _ohem(target, predict)


# speedup vs baseline: 3.8537x; 3.8537x over previous
import jax
import jax.numpy as jnp
from jax import lax
from jax.experimental import pallas as pl
from jax.experimental.pallas import tpu as pltpu

N_ROIS = 20000
LOSS_DIM = 84
KEEP = 512
CHUNK = 2000
NCHUNK = N_ROIS // CHUNK
F32_INF_BITS = 0x7F800000


def _ohem_body(t_hbm, p_hbm, out_ref, tbuf, pbuf, loss_ref, tsem, psem):
    for c in range(NCHUNK):
        sl = pl.ds(c * CHUNK, CHUNK)
        pltpu.make_async_copy(t_hbm.at[sl, :], tbuf.at[sl, :], tsem.at[c]).start()
        pltpu.make_async_copy(p_hbm.at[sl, :], pbuf.at[sl, :], psem.at[c]).start()

    ones = jnp.ones((1, LOSS_DIM), dtype=jnp.float32)
    for c in range(NCHUNK):
        sl = pl.ds(c * CHUNK, CHUNK)
        pltpu.make_async_copy(t_hbm.at[sl, :], tbuf.at[sl, :], tsem.at[c]).wait()
        pltpu.make_async_copy(p_hbm.at[sl, :], pbuf.at[sl, :], psem.at[c]).wait()
        d = jnp.abs(tbuf[sl, :] - pbuf[sl, :])
        l = jnp.where(d < 1.0, 0.5 * d * d, d - 0.5)
        row = lax.dot_general(
            ones, l,
            dimension_numbers=(((1,), (1,)), ((), ())),
            precision=lax.Precision.DEFAULT,
            preferred_element_type=jnp.float32,
        )
        loss_ref[c, :] = row[0, :]

    vals = loss_ref[...]
    bits = lax.bitcast_convert_type(vals, jnp.int32)

    def count_ge(m):
        return jnp.sum(jnp.where(bits >= m, 1, 0))

    def body(_, carry):
        lo, hi = carry
        q = jnp.maximum((hi - lo) // 4, 1)
        m1 = lo + q
        m2 = lo + 2 * q
        m3 = lo + 3 * q
        c1 = count_ge(m1) >= KEEP
        c2 = count_ge(m2) >= KEEP
        c3 = count_ge(m3) >= KEEP
        lo2 = jnp.where(c3, m3, jnp.where(c2, m2, jnp.where(c1, m1, lo)))
        hi2 = jnp.where(c1, jnp.where(c2, jnp.where(c3, hi, m3), m2), m1)
        return lo2, hi2

    lo, hi = lax.fori_loop(
        0, 16, body, (jnp.int32(0), jnp.int32(F32_INF_BITS)))
    t_val = lax.bitcast_convert_type(lo, jnp.float32)
    gt = bits > lo
    cnt_gt = jnp.sum(jnp.where(gt, 1, 0))
    sum_gt = jnp.sum(jnp.where(gt, vals, 0.0))
    res = (sum_gt + (KEEP - cnt_gt).astype(jnp.float32) * t_val) / KEEP
    out_ref[0, 0] = res


@jax.jit
def _ohem(target, predict):
    out = pl.pallas_call(
        _ohem_body,
        in_specs=[
            pl.BlockSpec(memory_space=pl.ANY),
            pl.BlockSpec(memory_space=pl.ANY),
        ],
        out_specs=pl.BlockSpec(memory_space=pltpu.SMEM),
        out_shape=jax.ShapeDtypeStruct((1, 1), jnp.float32),
        scratch_shapes=[
            pltpu.VMEM((N_ROIS, LOSS_DIM), jnp.float32),
            pltpu.VMEM((N_ROIS, LOSS_DIM), jnp.float32),
            pltpu.VMEM((NCHUNK, CHUNK), jnp.float32),
            pltpu.SemaphoreType.DMA((NCHUNK,)),
            pltpu.SemaphoreType.DMA((NCHUNK,)),
        ],
    )(target, predict)
    return out[0, 0]


def kernel(target, predict):
    return _ohem(target, predict)
